# baseline (speedup 1.0000x reference)
import functools

import jax
import jax.numpy as jnp
import numpy as np
from jax.experimental import pallas as pl

H = 128
NRBF = 32
NL = 4
ANF = 16
RNF = 21
NV = 8
CUT = 8.0


def _ln(x, g, b):
    mu = jnp.mean(x, axis=-1, keepdims=True)
    var = jnp.var(x, axis=-1, keepdims=True)
    return (x - mu) / jnp.sqrt(var + 1e-5) * g + b


def _enc(x, p):
    h = jax.nn.silu(_ln(x @ p['w1'] + p['b1'], p['g1'], p['bb1']))
    return _ln(h @ p['w2'] + p['b2'], p['g2'], p['bb2'])


def _geb(x, v, p, scalar_act):
    vec1 = jnp.einsum('nvc,cd->nvd', v, p['wv1'])
    vec1n = jnp.sqrt(jnp.sum(vec1 * vec1, axis=1) + 1e-12)
    vec2 = jnp.einsum('nvc,cd->nvd', v, p['wv2'])
    h = jnp.concatenate([x, vec1n], axis=-1)
    h = jax.nn.silu(h @ p['u1w'] + p['u1b'])
    h = h @ p['u2w'] + p['u2b']
    dout = p['wv2'].shape[1]
    xo, gate = h[:, :dout], h[:, dout:]
    vo = vec2 * gate[:, None, :]
    if scalar_act:
        xo = jax.nn.silu(xo)
    return xo, vo


def _edge_geom_pallas(diff, etype, etype_emb):
    E = diff.shape[0]
    B = 1920
    means = jnp.linspace(float(np.exp(-CUT)), 1.0, NRBF)
    beta = float(((2.0 / NRBF) * (1.0 - np.exp(-CUT))) ** -2)

    def body(diff_ref, et_ref, emb_ref, means_ref, sph_ref, efb_ref, c_ref):
        dif = diff_ref[...]
        d = jnp.sqrt(jnp.sum(dif * dif, axis=-1, keepdims=True) + 1e-12)
        u = dif / d
        ux, uy, uz = u[:, 0:1], u[:, 1:2], u[:, 2:3]
        d1 = d[:, 0]
        sph = jnp.concatenate([
            ux, uy, uz, ux * uy, uy * uz, uz * ux,
            ux * ux - uy * uy, 3.0 * uz * uz - 1.0], axis=-1)
        rbf = jnp.exp(-beta * (jnp.exp(-d) - means_ref[...][0][None, :]) ** 2)
        C = jnp.where(d1 < CUT, 0.5 * (jnp.cos(jnp.pi * d1 / CUT) + 1.0), 0.0)
        et = et_ref[...][:, 0]
        emb = emb_ref[...]
        emb_sel = (jnp.where((et == 0)[:, None], emb[0][None, :], 0.0)
                   + jnp.where((et == 1)[:, None], emb[1][None, :], 0.0)
                   + jnp.where((et == 2)[:, None], emb[2][None, :], 0.0))
        efb_ref[...] = rbf * C[:, None] + emb_sel
        sph_ref[...] = sph
        c_ref[...] = C[:, None]

    sph, efb, c = pl.pallas_call(
        body,
        grid=(E // B,),
        in_specs=[
            pl.BlockSpec((B, 3), lambda i: (i, 0)),
            pl.BlockSpec((B, 1), lambda i: (i, 0)),
            pl.BlockSpec((3, NRBF), lambda i: (0, 0)),
            pl.BlockSpec((1, NRBF), lambda i: (0, 0)),
        ],
        out_specs=[
            pl.BlockSpec((B, NV), lambda i: (i, 0)),
            pl.BlockSpec((B, NRBF), lambda i: (i, 0)),
            pl.BlockSpec((B, 1), lambda i: (i, 0)),
        ],
        out_shape=[
            jax.ShapeDtypeStruct((E, NV), jnp.float32),
            jax.ShapeDtypeStruct((E, NRBF), jnp.float32),
            jax.ShapeDtypeStruct((E, 1), jnp.float32),
        ],
    )(diff, etype[:, None], etype_emb, means[None, :])
    return sph, efb, c[:, 0]


def kernel(xh_atoms, xh_residues, t, mask_atoms, mask_residues, edge_index, edge_types, params):
    na = xh_atoms.shape[0]
    n = na + xh_residues.shape[0]
    pos = jnp.concatenate([xh_atoms[:, :3], xh_residues[:, :3]], axis=0)
    h = jnp.concatenate([_enc(xh_atoms[:, 3:], params['atom_enc']),
                         _enc(xh_residues[:, 3:], params['res_enc'])], axis=0)
    half = H // 2
    freq = jnp.exp(jnp.arange(half, dtype=jnp.float32) * (-np.log(10000.0) / (half - 1)))
    te = t[:, None] * freq[None, :]
    te = jnp.concatenate([jnp.sin(te), jnp.cos(te)], axis=-1) @ params['time_w'] + params['time_b']
    mask = jnp.concatenate([mask_atoms, mask_residues], axis=0)
    h = jnp.concatenate([h, te[mask]], axis=-1)
    x = h @ params['in_w'] + params['in_b']
    v = jnp.zeros((n, NV, H), jnp.float32)
    src = edge_index[0]
    dst = edge_index[1]
    diff = pos[dst] - pos[src]

    sph, ef_base, C = _edge_geom_pallas(diff, edge_types, params['etype_emb'])

    for lp in params['layers']:
        ef = jax.nn.silu(ef_base @ lp['we'] + lp['be'])
        m = jax.nn.silu((x[src] + ef) @ lp['wm'] + lp['bm']) * C[:, None]
        x = x + jax.ops.segment_sum(m, dst, num_segments=n)
        dv = jnp.stack([jax.ops.segment_sum(sph[:, i:i + 1] * m, dst, num_segments=n) for i in range(NV)], axis=1)
        v = v + dv
        x = x + jax.nn.silu(x @ lp['wu'] + lp['bu'])
    cp = params['coord']
    xc, vc = _geb(x, v, cp['g1'], True)
    xc, vc = _geb(xc, vc, cp['g2'], True)
    v_l1 = vc[:, :3, :]
    v_l2 = vc[:, 3:8, :]
    l1v = jnp.einsum('nvc,co->nvo', v_l1, cp['l1_w'])[:, :, 0] + cp['l1_b']
    l2m = jnp.sqrt(jnp.sum(v_l2 * v_l2, axis=1) + 1e-12)
    l2mod = jnp.tanh(jax.nn.silu(l2m @ cp['l2_w1'] + cp['l2_b1']) @ cp['l2_w2'] + cp['l2_b2'])
    mag = jax.nn.sigmoid(jax.nn.silu(xc @ cp['sn_w1'] + cp['sn_b1']) @ cp['sn_w2'] + cp['sn_b2'])
    w = jax.nn.softmax(cp['comb'])
    vel = mag * ((w[0] + w[1] * l2mod) * l1v)
    fa = params['feat_a']
    fr = params['feat_r']
    xa, va = _geb(x[:na], v[:na], fa['g1'], True)
    xa, _ = _geb(xa, va, fa['g2'], False)
    xr, vr = _geb(x[na:], v[na:], fr['g1'], True)
    xr, _ = _geb(xr, vr, fr['g2'], False)
    return (jnp.concatenate([vel[:na], xa], axis=-1), jnp.concatenate([vel[na:], xr], axis=-1))


# speedup vs baseline: 2.3958x; 2.3958x over previous
import functools

import jax
import jax.numpy as jnp
import numpy as np
from jax import lax
from jax.experimental import pallas as pl
from jax.experimental.pallas import tpu as pltpu
from jax.experimental.pallas import tpu_sc as plsc

H = 128
NRBF = 32
NL = 4
ANF = 16
RNF = 21
NV = 8
CUT = 8.0
NA = 10000
NRES = 2000
N = NA + NRES
E = 192000
NW = 32
K = 240
EB = 1920
NB = 1000
NPAD = 12032

_MESH = plsc.VectorSubcoreMesh(core_axis_name="c", subcore_axis_name="s")


def _silu(x):
    return x * jax.nn.sigmoid(x)


def _ln(h, g, b):
    mu = jnp.mean(h, axis=-1, keepdims=True)
    va = jnp.mean((h - mu) ** 2, axis=-1, keepdims=True)
    return (h - mu) / jnp.sqrt(va + 1e-5) * g + b


def _tc_encoder(xf, p, in_wh, in_wt, in_b, wm0, t, tw, tb, brows):
    nrows, din = xf.shape
    half = H // 2
    freq = jnp.exp(jnp.arange(half, dtype=jnp.float32)
                   * (-np.log(10000.0) / (half - 1)))[None, :]

    def body(x_ref, w1, b1, g1, bb1, w2, b2, g2, bb2, inw, inwt, inb, wm,
             t_ref, fr_ref, tw_ref, tb_ref, x0_ref, xw_ref):
        x = x_ref[...]
        h = _silu(_ln(x @ w1[...] + b1[...], g1[...], bb1[...]))
        h = _ln(h @ w2[...] + b2[...], g2[...], bb2[...])
        te_arg = t_ref[...] * fr_ref[...]
        te = (jnp.sum(jnp.sin(te_arg) * tw_ref[0:1, :])
              + jnp.sum(jnp.cos(te_arg) * tw_ref[1:2, :]) + tb_ref[0, 0])
        x0 = h @ inw[...] + te * inwt[...] + inb[...]
        x0_ref[...] = x0
        xw_ref[...] = x0 @ wm[...]

    full = lambda a, b: pl.BlockSpec((a, b), lambda i: (0, 0))
    return pl.pallas_call(
        body,
        grid=(nrows // brows,),
        in_specs=[
            pl.BlockSpec((brows, din), lambda i: (i, 0)),
            full(din, half), full(1, half), full(1, half), full(1, half),
            full(half, H), full(1, H), full(1, H), full(1, H),
            full(H, H), full(1, H), full(1, H), full(H, H),
            full(1, 1), full(1, half), full(2, half), full(1, 1),
        ],
        out_specs=[
            pl.BlockSpec((brows, H), lambda i: (i, 0)),
            pl.BlockSpec((brows, H), lambda i: (i, 0)),
        ],
        out_shape=[
            jax.ShapeDtypeStruct((nrows, H), jnp.float32),
            jax.ShapeDtypeStruct((nrows, H), jnp.float32),
        ],
    )(xf, p['w1'], p['b1'][None], p['g1'][None], p['bb1'][None],
      p['w2'], p['b2'][None], p['g2'][None], p['bb2'][None],
      in_wh, in_wt, in_b, wm0, t[:, None], freq, tw, tb)


def _sc_pos_diff(pos128, src, dst):
    @functools.partial(
        pl.kernel,
        mesh=_MESH,
        out_type=jax.ShapeDtypeStruct((E, 128), jnp.float32),
        scratch_types=[
            pltpu.VMEM((K,), jnp.int32),
            pltpu.VMEM((K,), jnp.int32),
            pltpu.VMEM((K, 128), jnp.float32),
            pltpu.VMEM((K, 128), jnp.float32),
            pltpu.SemaphoreType.DMA,
            pltpu.SemaphoreType.DMA,
        ],
    )
    def k(pos_hbm, src_hbm, dst_hbm, out_hbm, si_v, di_v, a_v, b_v, s1, s2):
        wid = lax.axis_index("s") * 2 + lax.axis_index("c")
        ept = E // NW
        nslab = ept // K

        def body(i, _):
            base = wid * ept + i * K
            pltpu.sync_copy(src_hbm.at[pl.ds(base, K)], si_v)
            pltpu.sync_copy(dst_hbm.at[pl.ds(base, K)], di_v)
            ca = pltpu.async_copy(pos_hbm.at[si_v], a_v, s1)
            cb = pltpu.async_copy(pos_hbm.at[di_v], b_v, s2)
            ca.wait()
            cb.wait()

            def row(r, _):
                b_v[r, pl.ds(0, 16)] = b_v[r, pl.ds(0, 16)] - a_v[r, pl.ds(0, 16)]
                return 0

            lax.fori_loop(0, K, row, 0, unroll=4)
            pltpu.sync_copy(b_v, out_hbm.at[pl.ds(base, K)])
            return 0

        lax.fori_loop(0, nslab, body, 0)

    return k(pos128, src, dst)


def _tc_edge_geom(diff16, etype, etype_emb):
    means = jnp.linspace(float(np.exp(-CUT)), 1.0, NRBF)[None, :]
    beta = float(((2.0 / NRBF) * (1.0 - np.exp(-CUT))) ** -2)

    def body(diff_ref, et_ref, emb_ref, means_ref, efb_ref, wts_ref):
        dif = diff_ref[...]
        dx, dy, dz = dif[:, 0:1], dif[:, 1:2], dif[:, 2:3]
        d = jnp.sqrt(dx * dx + dy * dy + dz * dz + 1e-12)
        ux, uy, uz = dx / d, dy / d, dz / d
        C = jnp.where(d < CUT, 0.5 * (jnp.cos(jnp.pi * d / CUT) + 1.0), 0.0)
        rbf = jnp.exp(-beta * (jnp.exp(-d) - means_ref[...]) ** 2)
        et = et_ref[...][:, 0]
        emb = emb_ref[...]
        emb_sel = (jnp.where((et == 0)[:, None], emb[0][None, :], 0.0)
                   + jnp.where((et == 1)[:, None], emb[1][None, :], 0.0)
                   + jnp.where((et == 2)[:, None], emb[2][None, :], 0.0))
        efb_ref[...] = rbf * C + emb_sel
        z = jnp.zeros_like(C)
        wts_ref[...] = jnp.concatenate([
            C, C * ux, C * uy, C * uz,
            C * ux * uy, C * uy * uz, C * uz * ux,
            C * (ux * ux - uy * uy), C * (3.0 * uz * uz - 1.0),
            z, z, z, z, z, z, z], axis=1)

    return pl.pallas_call(
        body,
        grid=(E // EB,),
        in_specs=[
            pl.BlockSpec((EB, 128), lambda i: (i, 0)),
            pl.BlockSpec((EB, 1), lambda i: (i, 0)),
            pl.BlockSpec((3, NRBF), lambda i: (0, 0)),
            pl.BlockSpec((1, NRBF), lambda i: (0, 0)),
        ],
        out_specs=[
            pl.BlockSpec((EB, NRBF), lambda i: (i, 0)),
            pl.BlockSpec((EB, 16), lambda i: (i, 0)),
        ],
        out_shape=[
            jax.ShapeDtypeStruct((E, NRBF), jnp.float32),
            jax.ShapeDtypeStruct((E, 16), jnp.float32),
        ],
    )(diff16, etype[:, None], etype_emb, means)


def _tc_edge_dense(efb, we, be, wm, bm):
    def body(efb_ref, we_r, be_r, wm_r, bm_r, out_ref):
        ef = _silu(efb_ref[...] @ we_r[...] + be_r[...])
        out_ref[...] = ef @ wm_r[...] + bm_r[...]

    return pl.pallas_call(
        body,
        grid=(E // EB,),
        in_specs=[
            pl.BlockSpec((EB, NRBF), lambda i: (i, 0)),
            pl.BlockSpec((NRBF, H), lambda i: (0, 0)),
            pl.BlockSpec((1, H), lambda i: (0, 0)),
            pl.BlockSpec((H, H), lambda i: (0, 0)),
            pl.BlockSpec((1, H), lambda i: (0, 0)),
        ],
        out_specs=pl.BlockSpec((EB, H), lambda i: (i, 0)),
        out_shape=jax.ShapeDtypeStruct((E, H), jnp.float32),
    )(efb, we, be[None], wm, bm[None])


def _sc_gather_silu(xw, efw, src):
    @functools.partial(
        pl.kernel,
        mesh=_MESH,
        out_type=jax.ShapeDtypeStruct((E, H), jnp.float32),
        scratch_types=[
            pltpu.VMEM((K,), jnp.int32),
            pltpu.VMEM((K, H), jnp.float32),
            pltpu.VMEM((K, H), jnp.float32),
            pltpu.SemaphoreType.DMA,
        ],
    )
    def k(xw_hbm, efw_hbm, src_hbm, out_hbm, idx_v, g_v, e_v, sem):
        wid = lax.axis_index("s") * 2 + lax.axis_index("c")
        ept = E // NW
        nslab = ept // K

        def body(i, _):
            base = wid * ept + i * K
            pltpu.sync_copy(src_hbm.at[pl.ds(base, K)], idx_v)
            pltpu.async_copy(xw_hbm.at[idx_v], g_v, sem).wait()
            pltpu.sync_copy(efw_hbm.at[pl.ds(base, K)], e_v)

            def row(r, _):
                for j in range(H // 16):
                    p = g_v[r, pl.ds(j * 16, 16)] + e_v[r, pl.ds(j * 16, 16)]
                    e_v[r, pl.ds(j * 16, 16)] = p / (1.0 + jnp.exp(-p))
                return 0

            lax.fori_loop(0, K, row, 0)
            pltpu.sync_copy(e_v, out_hbm.at[pl.ds(base, K)])
            return 0

        lax.fori_loop(0, nslab, body, 0)

    return k(xw, efw, src)


def _sc_scatter9(sil, wts_flat, dst):
    KS = 160

    @functools.partial(
        pl.kernel,
        mesh=_MESH,
        out_type=jax.ShapeDtypeStruct((9 * NPAD, H), jnp.float32),
        scratch_types=[
            pltpu.VMEM((KS,), jnp.int32),
            pltpu.VMEM((KS,), jnp.float32),
            pltpu.VMEM((KS, H), jnp.float32),
            pltpu.VMEM((16, H), jnp.float32),
            pltpu.VMEM_SHARED((NPAD, H), jnp.float32),
        ],
    )
    def k(sil_hbm, wts_hbm, dst_hbm, out_hbm, didx_v, w_v, m_v, z_v, acc):
        cid = lax.axis_index("c")
        sid = lax.axis_index("s")
        ept = E // 16
        nslab = ept // KS
        trows = NPAD // 16

        for r in range(16):
            for j in range(H // 16):
                z_v[r, pl.ds(j * 16, 16)] = jnp.zeros((16,), jnp.float32)

        def one_pass(p, _):
            ci = p * 2 + cid

            @pl.when(ci < 9)
            def _():
                def zbody(i, _):
                    pltpu.sync_copy(z_v, acc.at[pl.ds(sid * trows + i * 16, 16)])
                    return 0

                lax.fori_loop(0, trows // 16, zbody, 0)
                plsc.subcore_barrier()

                def body(i, _):
                    base = sid * ept + i * KS
                    pltpu.sync_copy(dst_hbm.at[pl.ds(base, KS)], didx_v)
                    pltpu.sync_copy(sil_hbm.at[pl.ds(base, KS)], m_v)
                    pltpu.sync_copy(wts_hbm.at[pl.ds(ci * E + base, KS)], w_v)

                    def grp(g, _):
                        w16 = w_v[pl.ds(g * 16, 16)]
                        for l in range(16):
                            spl = w16.at[jnp.full((16,), l, jnp.int32)].get(
                                mode='promise_in_bounds')
                            for j in range(H // 16):
                                m_v[g * 16 + l, pl.ds(j * 16, 16)] = (
                                    m_v[g * 16 + l, pl.ds(j * 16, 16)] * spl)
                        return 0

                    lax.fori_loop(0, KS // 16, grp, 0)
                    pltpu.sync_copy(m_v, acc.at[didx_v], add=True)
                    return 0

                lax.fori_loop(0, nslab, body, 0)
                plsc.subcore_barrier()
                pltpu.sync_copy(acc.at[pl.ds(sid * trows, trows)],
                                out_hbm.at[pl.ds(ci * NPAD + sid * trows, trows)])
                plsc.subcore_barrier()

            return 0

        lax.fori_loop(0, 5, one_pass, 0)

    return k(sil, wts_flat, dst)


def _tc_node_update(x, dv0, wu, bu, wm_next):
    def body(x_ref, dv_ref, wu_r, bu_r, wm_r, xo_ref, xw_ref):
        y = x_ref[...] + dv_ref[...]
        y = y + _silu(y @ wu_r[...] + bu_r[...])
        xo_ref[...] = y
        xw_ref[...] = y @ wm_r[...]

    return pl.pallas_call(
        body,
        grid=(N // NB,),
        in_specs=[
            pl.BlockSpec((NB, H), lambda i: (i, 0)),
            pl.BlockSpec((NB, H), lambda i: (i, 0)),
            pl.BlockSpec((H, H), lambda i: (0, 0)),
            pl.BlockSpec((1, H), lambda i: (0, 0)),
            pl.BlockSpec((H, H), lambda i: (0, 0)),
        ],
        out_specs=[
            pl.BlockSpec((NB, H), lambda i: (i, 0)),
            pl.BlockSpec((NB, H), lambda i: (i, 0)),
        ],
        out_shape=[
            jax.ShapeDtypeStruct((N, H), jnp.float32),
            jax.ShapeDtypeStruct((N, H), jnp.float32),
        ],
    )(x, dv0, wu, bu[None], wm_next)


def _geb_blk(x, v, wv1, wv2, u1w, u1b, u2w, u2b, dout, scalar_act):
    nv, nb, din = v.shape
    vec1 = (v.reshape(nv * nb, din) @ wv1).reshape(nv, nb, din)
    vec1n = jnp.sqrt(jnp.sum(vec1 * vec1, axis=0) + 1e-12)
    vec2 = (v.reshape(nv * nb, din) @ wv2).reshape(nv, nb, wv2.shape[1])
    h = jnp.concatenate([x, vec1n], axis=-1)
    h = _silu(h @ u1w + u1b)
    h = h @ u2w + u2b
    xo, gate = h[:, :dout], h[:, dout:]
    vo = vec2 * gate[None, :, :]
    if scalar_act:
        xo = _silu(xo)
    return xo, vo


def _tc_head(x, dvv, cw, fw):
    def body(x_ref, d1, d2, d3, d4,
             c_wv1, c_wv2, c_u1w, c_u1b, c_u2w, c_u2b,
             c2_wv1, c2_wv2, c2_u1w, c2_u1b, c2_u2w, c2_u2b,
             sn_w1, sn_b1, sn_w2, sn_b2, l1_w, l1_b,
             l2_w1, l2_b1, l2_w2, l2_b2, comb,
             f_wv1, f_wv2, f_u1w, f_u1b, f_u2w, f_u2b,
             f2_wv1, f2_wv2, f2_u1w, f2_u1b, f2_u2w, f2_u2b,
             vel_ref, xf_ref):
        x = x_ref[...]
        v = d1[...] + d2[...] + d3[...] + d4[...]
        fsel0 = lambda r: r[...][0]
        xc, vc = _geb_blk(x, v, c_wv1[...], c_wv2[...], c_u1w[...],
                          c_u1b[...], c_u2w[...], c_u2b[...], H // 2, True)
        xc, vc = _geb_blk(xc, vc, c2_wv1[...], c2_wv2[...], c2_u1w[...],
                          c2_u1b[...], c2_u2w[...], c2_u2b[...], H // 4, True)
        l1w = l1_w[...]
        l1v = jnp.concatenate(
            [jnp.sum(vc[kk] * l1w, axis=-1, keepdims=True) for kk in range(3)],
            axis=1) + l1_b[...]
        vl2 = vc[3:8]
        l2m = jnp.sqrt(jnp.sum(vl2 * vl2, axis=0) + 1e-12)
        l2mod = jnp.tanh(_silu(l2m @ l2_w1[...] + l2_b1[...]) @ l2_w2[...] + l2_b2[...])
        mag = jax.nn.sigmoid(_silu(xc @ sn_w1[...] + sn_b1[...]) @ sn_w2[...] + sn_b2[...])
        c0 = comb[0, 0]
        c1 = comb[0, 1]
        e0 = jnp.exp(c0 - jnp.maximum(c0, c1))
        e1 = jnp.exp(c1 - jnp.maximum(c0, c1))
        w0 = e0 / (e0 + e1)
        w1 = e1 / (e0 + e1)
        vel = mag * ((w0 + w1 * l2mod) * l1v)
        vel_ref[...] = jnp.concatenate([vel, jnp.zeros((vel.shape[0], 5), jnp.float32)], axis=1)
        xa, va = _geb_blk(x, v, fsel0(f_wv1), fsel0(f_wv2), fsel0(f_u1w),
                          fsel0(f_u1b), fsel0(f_u2w), fsel0(f_u2b), H // 2, True)
        xa, _ = _geb_blk(xa, va, fsel0(f2_wv1), fsel0(f2_wv2), fsel0(f2_u1w),
                         fsel0(f2_u1b), fsel0(f2_u2w), fsel0(f2_u2b), RNF, False)
        xf_ref[...] = xa

    full = lambda *s: pl.BlockSpec(s, lambda i: (0,) * len(s))
    fsel = lambda *s: pl.BlockSpec((1,) + s[1:],
                                   lambda i: (i // 10,) + (0,) * (len(s) - 1))
    dspec = pl.BlockSpec((NV, NB, H), lambda i: (0, i, 0))
    vel, xf = pl.pallas_call(
        body,
        grid=(N // NB,),
        in_specs=[
            pl.BlockSpec((NB, H), lambda i: (i, 0)),
            dspec, dspec, dspec, dspec,
            full(H, H), full(H, H // 2), full(2 * H, H), full(1, H),
            full(H, H), full(1, H),
            full(H // 2, H // 2), full(H // 2, H // 4), full(H, H // 2),
            full(1, H // 2), full(H // 2, H // 2), full(1, H // 2),
            full(H // 4, H // 8), full(1, H // 8), full(H // 8, 1), full(1, 1),
            full(1, H // 4), full(1, 1),
            full(H // 4, H // 8), full(1, H // 8), full(H // 8, 1), full(1, 1),
            full(1, 2),
            fsel(2, H, H), fsel(2, H, H // 2), fsel(2, 2 * H, H), fsel(2, 1, H),
            fsel(2, H, H), fsel(2, 1, H),
            fsel(2, H // 2, H // 2), fsel(2, H // 2, RNF), fsel(2, H, H // 2),
            fsel(2, 1, H // 2), fsel(2, H // 2, 2 * RNF), fsel(2, 1, 2 * RNF),
        ],
        out_specs=[
            pl.BlockSpec((NB, 8), lambda i: (i, 0)),
            pl.BlockSpec((NB, RNF), lambda i: (i, 0)),
        ],
        out_shape=[
            jax.ShapeDtypeStruct((N, 8), jnp.float32),
            jax.ShapeDtypeStruct((N, RNF), jnp.float32),
        ],
    )(x, dvv[0], dvv[1], dvv[2], dvv[3], *cw, *fw)
    return vel, xf


def _pad_cols(a, w):
    return jnp.pad(a, ((0, 0), (0, w - a.shape[1])))


def kernel(xh_atoms, xh_residues, t, mask_atoms, mask_residues, edge_index, edge_types, params):
    src = edge_index[0].astype(jnp.int32)
    dst = edge_index[1].astype(jnp.int32)
    etype = edge_types.astype(jnp.int32)
    lp = params['layers']
    in_w = params['in_w']
    tw = jnp.stack([params['time_w'][:H // 2, 0], params['time_w'][H // 2:, 0]])
    tb = params['time_b'][None]

    x0a, xwa = _tc_encoder(xh_atoms[:, 3:], params['atom_enc'], in_w[:H],
                           in_w[H:H + 1], params['in_b'][None], lp[0]['wm'],
                           t, tw, tb, 2000)
    x0r, xwr = _tc_encoder(xh_residues[:, 3:], params['res_enc'], in_w[:H],
                           in_w[H:H + 1], params['in_b'][None], lp[0]['wm'],
                           t, tw, tb, 2000)
    x = jnp.concatenate([x0a, x0r], axis=0)
    xw = jnp.concatenate([xwa, xwr], axis=0)

    pos128 = jnp.zeros((N, 128), jnp.float32).at[:, :3].set(
        jnp.concatenate([xh_atoms[:, :3], xh_residues[:, :3]], axis=0))
    diff = _sc_pos_diff(pos128, src, dst)
    efb, wts_e = _tc_edge_geom(diff, etype, params['etype_emb'])
    wts_flat = wts_e.T.reshape(-1)

    dvvs = []
    for li in range(NL):
        p = lp[li]
        efw = _tc_edge_dense(efb, p['we'], p['be'], p['wm'], p['bm'])
        sil = _sc_gather_silu(xw, efw, src)
        dv9 = _sc_scatter9(sil, wts_flat, dst).reshape(9, NPAD, H)
        dvvs.append(dv9[1:9, :N])
        wm_next = lp[li + 1]['wm'] if li + 1 < NL else p['wu']
        x, xw = _tc_node_update(x, dv9[0, :N], p['wu'], p['bu'], wm_next)

    cp = params['coord']
    cw = []
    for g, dout in ((cp['g1'], H // 2), (cp['g2'], H // 4)):
        cw += [g['wv1'], g['wv2'], g['u1w'], g['u1b'][None], g['u2w'], g['u2b'][None]]
    cw += [cp['sn_w1'], cp['sn_b1'][None], cp['sn_w2'], cp['sn_b2'][None],
           cp['l1_w'].T, cp['l1_b'][None],
           cp['l2_w1'], cp['l2_b1'][None], cp['l2_w2'], cp['l2_b2'][None],
           cp['comb'][None]]

    fa, fr = params['feat_a'], params['feat_r']
    fw = []
    for key in ('wv1', 'wv2', 'u1w', 'u1b', 'u2w', 'u2b'):
        a, r = fa['g1'][key], fr['g1'][key]
        if a.ndim == 1:
            a, r = a[None], r[None]
        fw.append(jnp.stack([a, r]))
    a2, r2 = fa['g2'], fr['g2']
    u2w_a = jnp.concatenate([_pad_cols(a2['u2w'][:, :ANF], RNF),
                             _pad_cols(a2['u2w'][:, ANF:], RNF)], axis=1)
    u2b_a = jnp.concatenate([jnp.pad(a2['u2b'][:ANF], (0, RNF - ANF)),
                             jnp.pad(a2['u2b'][ANF:], (0, RNF - ANF))])
    for key, aw, rw in (('wv1', a2['wv1'], r2['wv1']),
                        ('wv2', _pad_cols(a2['wv2'], RNF), r2['wv2']),
                        ('u1w', a2['u1w'], r2['u1w']),
                        ('u1b', a2['u1b'][None], r2['u1b'][None]),
                        ('u2w', u2w_a, r2['u2w']),
                        ('u2b', u2b_a[None], r2['u2b'][None])):
        fw.append(jnp.stack([aw, rw]))

    vel, xf = _tc_head(x, dvvs, cw, fw)
    out_a = jnp.concatenate([vel[:NA, :3], xf[:NA, :ANF]], axis=-1)
    out_r = jnp.concatenate([vel[NA:, :3], xf[NA:, :RNF]], axis=-1)
    return (out_a, out_r)
